# baseline (speedup 1.0000x reference)
import functools

import jax
import jax.numpy as jnp
from jax import lax
from jax.experimental import pallas as pl
from jax.experimental.pallas import tpu as pltpu
from jax.experimental.pallas import tpu_sc as plsc

N = 10000
E = 160000
D = 256

NC = 2
NS = 16
NQ = 4
HQ = D // NQ
N_PAD = 10240
E_PAD = 163840
CA = 128
KA = E_PAD // (NS * CA)
CD = 128
KD = E_PAD // (NC * NS * CD)
RPT = N_PAD // NS
NBUF = 4
RB = 1024
NBLK = N_PAD // RB

_mesh = plsc.VectorSubcoreMesh(core_axis_name="c", subcore_axis_name="s",
                               num_cores=NC, num_subcores=NS)
_sc_params = pltpu.CompilerParams(use_tc_tiling_on_sc=False)


def _deg_body(dstd, ones_hbm, zeros_hbm, out, idx_v, ones_v, sem, acc):
    c = lax.axis_index("c")
    s = lax.axis_index("s")
    pltpu.sync_copy(ones_hbm, ones_v)
    pltpu.sync_copy(zeros_hbm, acc.at[pl.ds(s * RPT, RPT)])
    plsc.subcore_barrier()
    pltpu.sync_copy(dstd.at[pl.ds((c * NS + s) * KD, KD)], idx_v)

    def step(i, carry):
        for b in range(4):
            j = i * 4 + b
            pltpu.async_copy(ones_v, acc.at[idx_v.at[j]], sem, add=True)
        for b in range(4):
            j = i * 4 + b
            pltpu.make_async_copy(ones_v, acc.at[idx_v.at[j]], sem).wait()
        return carry

    lax.fori_loop(0, KD // 4, step, 0)
    plsc.subcore_barrier()
    pltpu.sync_copy(acc.at[pl.ds(s * RPT, RPT)],
                    out.at[pl.ds(c * N_PAD + s * RPT, RPT)])


_deg_call = functools.partial(
    pl.kernel,
    out_type=jax.ShapeDtypeStruct((NC * N_PAD, 16), jnp.float32),
    mesh=_mesh,
    compiler_params=_sc_params,
    scratch_types=[
        pltpu.VMEM((KD, CD), jnp.int32),
        pltpu.VMEM((CD, 16), jnp.float32),
        pltpu.SemaphoreType.DMA,
        pltpu.VMEM_SHARED((N_PAD, 16), jnp.float32),
    ],
)(_deg_body)


def _agg_body(yflat, srcx, dstx, zeros_hbm, out, sidx_v, didx_v, rows_v, acc,
              *sems):
    gsems = sems[:NBUF]
    ssems = sems[NBUF:]
    c = lax.axis_index("c")
    s = lax.axis_index("s")

    pltpu.sync_copy(dstx.at[pl.ds(s * KA, KA)], didx_v)

    def fire_gather(j, b):
        pltpu.async_copy(yflat.at[sidx_v.at[j]], rows_v.at[b], gsems[b])

    def wait_gather(j, b):
        pltpu.make_async_copy(yflat.at[sidx_v.at[j]], rows_v.at[b],
                              gsems[b]).wait()

    def fire_scatter(j, b):
        pltpu.async_copy(rows_v.at[b], acc.at[didx_v.at[j]], ssems[b],
                         add=True)

    def wait_scatter(j, b):
        pltpu.make_async_copy(rows_v.at[b], acc.at[didx_v.at[j]],
                              ssems[b]).wait()

    for p in range(NQ // NC):
        q = c * (NQ // NC) + p
        pltpu.sync_copy(zeros_hbm, acc.at[pl.ds(s * RPT, RPT)])
        pltpu.sync_copy(srcx.at[pl.ds((q * NS + s) * KA, KA)], sidx_v)
        plsc.subcore_barrier()

        for b in range(NBUF):
            fire_gather(b, b)
        steps = KA // NBUF

        def step(i, carry):
            for b in range(NBUF):
                j = i * NBUF + b
                wait_gather(j, b)
                fire_scatter(j, b)
            for b in range(NBUF):
                j = i * NBUF + b
                wait_scatter(j, b)
                fire_gather(j + NBUF, b)
            return carry

        lax.fori_loop(0, steps - 1, step, 0)
        for b in range(NBUF):
            j = (steps - 1) * NBUF + b
            wait_gather(j, b)
            fire_scatter(j, b)
        for b in range(NBUF):
            j = (steps - 1) * NBUF + b
            wait_scatter(j, b)
        plsc.subcore_barrier()
        pltpu.sync_copy(acc.at[pl.ds(s * RPT, RPT)],
                        out.at[pl.ds(q * N_PAD + s * RPT, RPT)])


_agg_call = functools.partial(
    pl.kernel,
    out_type=jax.ShapeDtypeStruct((NQ * N_PAD, HQ), jnp.float32),
    mesh=_mesh,
    compiler_params=_sc_params,
    scratch_types=[
        pltpu.VMEM((KA, CA), jnp.int32),
        pltpu.VMEM((KA, CA), jnp.int32),
        pltpu.VMEM((NBUF, CA, HQ), jnp.float32),
        pltpu.VMEM_SHARED((N_PAD, HQ), jnp.float32),
    ] + [pltpu.SemaphoreType.DMA] * (2 * NBUF),
)(_agg_body)


def _dis(d0_ref, d1_ref):
    deg = d0_ref[:, 0] + d1_ref[:, 0] + 1.0
    return lax.rsqrt(deg)


def _pre_kernel(d0, d1, x_ref, w_ref, o_ref):
    dis = _dis(d0, d1)
    xw = jnp.dot(x_ref[...], w_ref[...], preferred_element_type=jnp.float32)
    for q in range(NQ):
        o_ref[q] = dis[:, None] * xw[:, q * HQ:(q + 1) * HQ]


def _mid_kernel(a0, a1, a2, a3, y0, y1, y2, y3, d0, d1, b_ref, w_ref, o_ref):
    dis = _dis(d0, d1)
    agg = jnp.concatenate([a0[0], a1[0], a2[0], a3[0]], axis=1)
    yy = jnp.concatenate([y0[0], y1[0], y2[0], y3[0]], axis=1)
    hid = jnp.maximum(dis[:, None] * (agg + yy) + b_ref[...][None, :], 0.0)
    hw = jnp.dot(hid, w_ref[...], preferred_element_type=jnp.float32)
    for q in range(NQ):
        o_ref[q] = dis[:, None] * hw[:, q * HQ:(q + 1) * HQ]


def _fin_kernel(a0, a1, a2, a3, y0, y1, y2, y3, d0, d1, b_ref, z_ref):
    dis = _dis(d0, d1)
    agg = jnp.concatenate([a0[0], a1[0], a2[0], a3[0]], axis=1)
    yy = jnp.concatenate([y0[0], y1[0], y2[0], y3[0]], axis=1)
    z_ref[...] = dis[:, None] * (agg + yy) + b_ref[...][None, :]


def _deg_specs():
    return [
        pl.BlockSpec((RB, 16), lambda i: (i, 0)),
        pl.BlockSpec((RB, 16), lambda i: (i + NBLK, 0)),
    ]


def _quarter_specs():
    return [
        pl.BlockSpec((1, RB, HQ), lambda i, k=k: (k, i, 0))
        for k in range(NQ)
    ]


_pre_call = pl.pallas_call(
    _pre_kernel,
    grid=(NBLK,),
    in_specs=_deg_specs() + [
        pl.BlockSpec((RB, D), lambda i: (i, 0)),
        pl.BlockSpec((D, D), lambda i: (0, 0)),
    ],
    out_specs=pl.BlockSpec((NQ, RB, HQ), lambda i: (0, i, 0)),
    out_shape=jax.ShapeDtypeStruct((NQ, N_PAD, HQ), jnp.float32),
)

_mid_call = pl.pallas_call(
    _mid_kernel,
    grid=(NBLK,),
    in_specs=_quarter_specs() + _quarter_specs() + _deg_specs() + [
        pl.BlockSpec((D,), lambda i: (0,)),
        pl.BlockSpec((D, D), lambda i: (0, 0)),
    ],
    out_specs=pl.BlockSpec((NQ, RB, HQ), lambda i: (0, i, 0)),
    out_shape=jax.ShapeDtypeStruct((NQ, N_PAD, HQ), jnp.float32),
)

_fin_call = pl.pallas_call(
    _fin_kernel,
    grid=(NBLK,),
    in_specs=_quarter_specs() + _quarter_specs() + _deg_specs() + [
        pl.BlockSpec((D,), lambda i: (0,)),
    ],
    out_specs=pl.BlockSpec((RB, D), lambda i: (i, 0)),
    out_shape=jax.ShapeDtypeStruct((N_PAD, D), jnp.float32),
)


def kernel(node_feature, edge_index, W1, b1, W2, b2):
    src = edge_index[0]
    dst = edge_index[1]
    pad = E_PAD - E
    trash = N + (jnp.arange(pad, dtype=jnp.int32) % (N_PAD - N))
    src_pad = jnp.concatenate([src, trash])
    dst_pad = jnp.concatenate([dst, trash])
    dstd = dst_pad.reshape(NC * NS * KD, CD)
    srcx = jnp.concatenate(
        [src_pad + q * N_PAD for q in range(NQ)]).reshape(NQ * NS * KA, CA)
    dstx = dst_pad.reshape(NS * KA, CA)
    xp = jnp.pad(node_feature, ((0, N_PAD - N), (0, 0)))
    ones16 = jnp.ones((CD, 16), jnp.float32)
    zeros16 = jnp.zeros((RPT, 16), jnp.float32)
    zerosQ = jnp.zeros((RPT, HQ), jnp.float32)

    deg2 = _deg_call(dstd, ones16, zeros16)

    y1 = _pre_call(deg2, deg2, xp, W1)
    agg1 = _agg_call(y1.reshape(NQ * N_PAD, HQ), srcx, dstx, zerosQ)
    a1 = agg1.reshape(NQ, N_PAD, HQ)
    y2 = _mid_call(a1, a1, a1, a1, y1, y1, y1, y1, deg2, deg2, b1, W2)
    agg2 = _agg_call(y2.reshape(NQ * N_PAD, HQ), srcx, dstx, zerosQ)
    a2 = agg2.reshape(NQ, N_PAD, HQ)
    z2 = _fin_call(a2, a2, a2, a2, y2, y2, y2, y2, deg2, deg2, b2)
    return z2[:N]


# speedup vs baseline: 1.0103x; 1.0103x over previous
import functools

import jax
import jax.numpy as jnp
from jax import lax
from jax.experimental import pallas as pl
from jax.experimental.pallas import tpu as pltpu
from jax.experimental.pallas import tpu_sc as plsc

N = 10000
E = 160000
D = 256

NC = 2
NS = 16
NQ = 4
HQ = D // NQ
N_PAD = 10240
E_PAD = 163840
CA = 128
KA = E_PAD // (NS * CA)
CD = 128
KD = E_PAD // (NC * NS * CD)
RPT = N_PAD // NS
NBUF = 8
RB = 1024
NBLK = N_PAD // RB

_mesh = plsc.VectorSubcoreMesh(core_axis_name="c", subcore_axis_name="s",
                               num_cores=NC, num_subcores=NS)
_sc_params = pltpu.CompilerParams(use_tc_tiling_on_sc=False)


def _deg_body(dstd, ones_hbm, zeros_hbm, out, idx_v, ones_v, sem, acc):
    c = lax.axis_index("c")
    s = lax.axis_index("s")
    pltpu.sync_copy(ones_hbm, ones_v)
    pltpu.sync_copy(zeros_hbm, acc.at[pl.ds(s * RPT, RPT)])
    plsc.subcore_barrier()
    pltpu.sync_copy(dstd.at[pl.ds((c * NS + s) * KD, KD)], idx_v)

    def step(i, carry):
        for b in range(4):
            j = i * 4 + b
            pltpu.async_copy(ones_v, acc.at[idx_v.at[j]], sem, add=True)
        for b in range(4):
            j = i * 4 + b
            pltpu.make_async_copy(ones_v, acc.at[idx_v.at[j]], sem).wait()
        return carry

    lax.fori_loop(0, KD // 4, step, 0)
    plsc.subcore_barrier()
    pltpu.sync_copy(acc.at[pl.ds(s * RPT, RPT)],
                    out.at[pl.ds(c * N_PAD + s * RPT, RPT)])


_deg_call = functools.partial(
    pl.kernel,
    out_type=jax.ShapeDtypeStruct((NC * N_PAD, 16), jnp.float32),
    mesh=_mesh,
    compiler_params=pltpu.CompilerParams(use_tc_tiling_on_sc=True),
    scratch_types=[
        pltpu.VMEM((KD, CD), jnp.int32),
        pltpu.VMEM((CD, 16), jnp.float32),
        pltpu.SemaphoreType.DMA,
        pltpu.VMEM_SHARED((N_PAD, 16), jnp.float32),
    ],
)(_deg_body)


def _agg_body(yflat, srcx, dstx, zeros_hbm, out, sidx_v, didx_v, rows_v, acc,
              *sems):
    gsems = sems[:NBUF]
    ssems = sems[NBUF:]
    c = lax.axis_index("c")
    s = lax.axis_index("s")

    pltpu.sync_copy(dstx.at[pl.ds(s * KA, KA)], didx_v)

    def fire_gather(j, b):
        pltpu.async_copy(yflat.at[sidx_v.at[j]], rows_v.at[b], gsems[b])

    def wait_gather(j, b):
        pltpu.make_async_copy(yflat.at[sidx_v.at[j]], rows_v.at[b],
                              gsems[b]).wait()

    def fire_scatter(j, b):
        pltpu.async_copy(rows_v.at[b], acc.at[didx_v.at[j]], ssems[b],
                         add=True)

    def wait_scatter(j, b):
        pltpu.make_async_copy(rows_v.at[b], acc.at[didx_v.at[j]],
                              ssems[b]).wait()

    for p in range(NQ // NC):
        q = c * (NQ // NC) + p
        pltpu.sync_copy(zeros_hbm, acc.at[pl.ds(s * RPT, RPT)])
        pltpu.sync_copy(srcx.at[pl.ds((q * NS + s) * KA, KA)], sidx_v)
        plsc.subcore_barrier()

        for b in range(NBUF):
            fire_gather(b, b)
        steps = KA // NBUF

        def step(i, carry):
            for b in range(NBUF):
                j = i * NBUF + b
                wait_gather(j, b)
                fire_scatter(j, b)
            for b in range(NBUF):
                j = i * NBUF + b
                wait_scatter(j, b)
                fire_gather(j + NBUF, b)
            return carry

        lax.fori_loop(0, steps - 1, step, 0)
        for b in range(NBUF):
            j = (steps - 1) * NBUF + b
            wait_gather(j, b)
            fire_scatter(j, b)
        for b in range(NBUF):
            j = (steps - 1) * NBUF + b
            wait_scatter(j, b)
        plsc.subcore_barrier()
        pltpu.sync_copy(acc.at[pl.ds(s * RPT, RPT)],
                        out.at[pl.ds(q * N_PAD + s * RPT, RPT)])


_agg_call = functools.partial(
    pl.kernel,
    out_type=jax.ShapeDtypeStruct((NQ * N_PAD, HQ), jnp.float32),
    mesh=_mesh,
    compiler_params=_sc_params,
    scratch_types=[
        pltpu.VMEM((KA, CA), jnp.int32),
        pltpu.VMEM((KA, CA), jnp.int32),
        pltpu.VMEM((NBUF, CA, HQ), jnp.float32),
        pltpu.VMEM_SHARED((N_PAD, HQ), jnp.float32),
    ] + [pltpu.SemaphoreType.DMA] * (2 * NBUF),
)(_agg_body)


def _dis(d0_ref, d1_ref):
    deg = d0_ref[:, 0] + d1_ref[:, 0] + 1.0
    return lax.rsqrt(deg)


def _pre_kernel(d0, d1, x_ref, w_ref, o_ref):
    dis = _dis(d0, d1)
    xw = jnp.dot(x_ref[...], w_ref[...], preferred_element_type=jnp.float32)
    for q in range(NQ):
        o_ref[q] = dis[:, None] * xw[:, q * HQ:(q + 1) * HQ]


def _mid_kernel(a0, a1, a2, a3, y0, y1, y2, y3, d0, d1, b_ref, w_ref, o_ref):
    dis = _dis(d0, d1)
    agg = jnp.concatenate([a0[0], a1[0], a2[0], a3[0]], axis=1)
    yy = jnp.concatenate([y0[0], y1[0], y2[0], y3[0]], axis=1)
    hid = jnp.maximum(dis[:, None] * (agg + yy) + b_ref[...][None, :], 0.0)
    hw = jnp.dot(hid, w_ref[...], preferred_element_type=jnp.float32)
    for q in range(NQ):
        o_ref[q] = dis[:, None] * hw[:, q * HQ:(q + 1) * HQ]


def _fin_kernel(a0, a1, a2, a3, y0, y1, y2, y3, d0, d1, b_ref, z_ref):
    dis = _dis(d0, d1)
    agg = jnp.concatenate([a0[0], a1[0], a2[0], a3[0]], axis=1)
    yy = jnp.concatenate([y0[0], y1[0], y2[0], y3[0]], axis=1)
    z_ref[...] = dis[:, None] * (agg + yy) + b_ref[...][None, :]


def _deg_specs():
    return [
        pl.BlockSpec((RB, 16), lambda i: (i, 0)),
        pl.BlockSpec((RB, 16), lambda i: (i + NBLK, 0)),
    ]


def _quarter_specs():
    return [
        pl.BlockSpec((1, RB, HQ), lambda i, k=k: (k, i, 0))
        for k in range(NQ)
    ]


_pre_call = pl.pallas_call(
    _pre_kernel,
    grid=(NBLK,),
    in_specs=_deg_specs() + [
        pl.BlockSpec((RB, D), lambda i: (i, 0)),
        pl.BlockSpec((D, D), lambda i: (0, 0)),
    ],
    out_specs=pl.BlockSpec((NQ, RB, HQ), lambda i: (0, i, 0)),
    out_shape=jax.ShapeDtypeStruct((NQ, N_PAD, HQ), jnp.float32),
)

_mid_call = pl.pallas_call(
    _mid_kernel,
    grid=(NBLK,),
    in_specs=_quarter_specs() + _quarter_specs() + _deg_specs() + [
        pl.BlockSpec((D,), lambda i: (0,)),
        pl.BlockSpec((D, D), lambda i: (0, 0)),
    ],
    out_specs=pl.BlockSpec((NQ, RB, HQ), lambda i: (0, i, 0)),
    out_shape=jax.ShapeDtypeStruct((NQ, N_PAD, HQ), jnp.float32),
)

_fin_call = pl.pallas_call(
    _fin_kernel,
    grid=(NBLK,),
    in_specs=_quarter_specs() + _quarter_specs() + _deg_specs() + [
        pl.BlockSpec((D,), lambda i: (0,)),
    ],
    out_specs=pl.BlockSpec((RB, D), lambda i: (i, 0)),
    out_shape=jax.ShapeDtypeStruct((N_PAD, D), jnp.float32),
)


def kernel(node_feature, edge_index, W1, b1, W2, b2):
    src = edge_index[0]
    dst = edge_index[1]
    pad = E_PAD - E
    trash = N + (jnp.arange(pad, dtype=jnp.int32) % (N_PAD - N))
    src_pad = jnp.concatenate([src, trash])
    dst_pad = jnp.concatenate([dst, trash])
    dstd = dst_pad.reshape(NC * NS * KD, CD)
    srcx = jnp.concatenate(
        [src_pad + q * N_PAD for q in range(NQ)]).reshape(NQ * NS * KA, CA)
    dstx = dst_pad.reshape(NS * KA, CA)
    xp = jnp.pad(node_feature, ((0, N_PAD - N), (0, 0)))
    ones16 = jnp.ones((CD, 16), jnp.float32)
    zeros16 = jnp.zeros((RPT, 16), jnp.float32)
    zerosQ = jnp.zeros((RPT, HQ), jnp.float32)

    deg2 = _deg_call(dstd, ones16, zeros16)

    y1 = _pre_call(deg2, deg2, xp, W1)
    agg1 = _agg_call(y1.reshape(NQ * N_PAD, HQ), srcx, dstx, zerosQ)
    a1 = agg1.reshape(NQ, N_PAD, HQ)
    y2 = _mid_call(a1, a1, a1, a1, y1, y1, y1, y1, deg2, deg2, b1, W2)
    agg2 = _agg_call(y2.reshape(NQ * N_PAD, HQ), srcx, dstx, zerosQ)
    a2 = agg2.reshape(NQ, N_PAD, HQ)
    z2 = _fin_call(a2, a2, a2, a2, y2, y2, y2, y2, deg2, deg2, b2)
    return z2[:N]


# speedup vs baseline: 1.0392x; 1.0286x over previous
import functools

import jax
import jax.numpy as jnp
from jax import lax
from jax.experimental import pallas as pl
from jax.experimental.pallas import tpu as pltpu
from jax.experimental.pallas import tpu_sc as plsc

N = 10000
E = 160000
D = 256

NC = 2
NS = 16
NQ = 4
HQ = D // NQ
N_PAD = 10240
E_PAD = 163840
CA = 128
KA = E_PAD // (NS * CA)
CD = 128
KD = E_PAD // (NC * NS * CD)
RPT = N_PAD // NS
NBUF = 8
RB = 1024
NBLK = N_PAD // RB

_mesh = plsc.VectorSubcoreMesh(core_axis_name="c", subcore_axis_name="s",
                               num_cores=NC, num_subcores=NS)
_sc_params = pltpu.CompilerParams(use_tc_tiling_on_sc=False)


def _deg_body(dstd, ones_hbm, zeros_hbm, out, idx_v, ones_v, sem, acc):
    c = lax.axis_index("c")
    s = lax.axis_index("s")
    pltpu.sync_copy(ones_hbm, ones_v)
    pltpu.sync_copy(zeros_hbm, acc.at[pl.ds(s * RPT, RPT)])
    plsc.subcore_barrier()
    pltpu.sync_copy(dstd.at[pl.ds((c * NS + s) * KD, KD)], idx_v)

    def step(i, carry):
        for b in range(4):
            j = i * 4 + b
            pltpu.async_copy(ones_v, acc.at[idx_v.at[j]], sem, add=True)
        for b in range(4):
            j = i * 4 + b
            pltpu.make_async_copy(ones_v, acc.at[idx_v.at[j]], sem).wait()
        return carry

    lax.fori_loop(0, KD // 4, step, 0)
    plsc.subcore_barrier()
    pltpu.sync_copy(acc.at[pl.ds(s * RPT, RPT)],
                    out.at[pl.ds(c * N_PAD + s * RPT, RPT)])


_deg_call = functools.partial(
    pl.kernel,
    out_type=jax.ShapeDtypeStruct((NC * N_PAD, 16), jnp.float32),
    mesh=_mesh,
    compiler_params=_sc_params,
    scratch_types=[
        pltpu.VMEM((KD, CD), jnp.int32),
        pltpu.VMEM((CD, 16), jnp.float32),
        pltpu.SemaphoreType.DMA,
        pltpu.VMEM_SHARED((N_PAD, 16), jnp.float32),
    ],
)(_deg_body)


def _agg_body(yflat, srcx, dstx, zeros_hbm, out, sidx_v, didx_v, rows_v, acc,
              *sems):
    gsems = sems[:NBUF]
    ssems = sems[NBUF:]
    c = lax.axis_index("c")
    s = lax.axis_index("s")

    pltpu.sync_copy(dstx.at[pl.ds(s * KA, KA)], didx_v)

    def fire_gather(j, b):
        pltpu.async_copy(yflat.at[sidx_v.at[j]], rows_v.at[b], gsems[b])

    def wait_gather(j, b):
        pltpu.make_async_copy(yflat.at[sidx_v.at[j]], rows_v.at[b],
                              gsems[b]).wait()

    def fire_scatter(j, b):
        pltpu.async_copy(rows_v.at[b], acc.at[didx_v.at[j]], ssems[b],
                         add=True)

    def wait_scatter(j, b):
        pltpu.make_async_copy(rows_v.at[b], acc.at[didx_v.at[j]],
                              ssems[b]).wait()

    for p in range(NQ // NC):
        q = c * (NQ // NC) + p
        pltpu.sync_copy(zeros_hbm, acc.at[pl.ds(s * RPT, RPT)])
        pltpu.sync_copy(srcx.at[pl.ds((q * NS + s) * KA, KA)], sidx_v)
        plsc.subcore_barrier()

        for b in range(NBUF):
            fire_gather(b, b)
        steps = KA // NBUF

        def step(i, carry):
            for b in range(NBUF):
                j = i * NBUF + b
                wait_gather(j, b)
                fire_scatter(j, b)
            for b in range(NBUF):
                j = i * NBUF + b
                wait_scatter(j, b)
                fire_gather(j + NBUF, b)
            return carry

        lax.fori_loop(0, steps - 1, step, 0)
        for b in range(NBUF):
            j = (steps - 1) * NBUF + b
            wait_gather(j, b)
            fire_scatter(j, b)
        for b in range(NBUF):
            j = (steps - 1) * NBUF + b
            wait_scatter(j, b)
        plsc.subcore_barrier()
        pltpu.sync_copy(acc.at[pl.ds(s * RPT, RPT)],
                        out.at[pl.ds(q * N_PAD + s * RPT, RPT)])


_agg_call = functools.partial(
    pl.kernel,
    out_type=jax.ShapeDtypeStruct((NQ * N_PAD, HQ), jnp.float32),
    mesh=_mesh,
    compiler_params=_sc_params,
    scratch_types=[
        pltpu.VMEM((KA, CA), jnp.int32),
        pltpu.VMEM((KA, CA), jnp.int32),
        pltpu.VMEM((NBUF, CA, HQ), jnp.float32),
        pltpu.VMEM_SHARED((N_PAD, HQ), jnp.float32),
    ] + [pltpu.SemaphoreType.DMA] * (2 * NBUF),
)(_agg_body)


def _dis(d0_ref, d1_ref):
    deg = d0_ref[:, 0] + d1_ref[:, 0] + 1.0
    return lax.rsqrt(deg)


def _pre_kernel(d0, d1, x_ref, w_ref, o_ref):
    dis = _dis(d0, d1)
    xw = jnp.dot(x_ref[...], w_ref[...], preferred_element_type=jnp.float32)
    for q in range(NQ):
        o_ref[q] = dis[:, None] * xw[:, q * HQ:(q + 1) * HQ]


def _mid_kernel(a0, a1, a2, a3, y0, y1, y2, y3, d0, d1, b_ref, w_ref, o_ref):
    dis = _dis(d0, d1)
    agg = jnp.concatenate([a0[0], a1[0], a2[0], a3[0]], axis=1)
    yy = jnp.concatenate([y0[0], y1[0], y2[0], y3[0]], axis=1)
    hid = jnp.maximum(dis[:, None] * (agg + yy) + b_ref[...][None, :], 0.0)
    hw = jnp.dot(hid, w_ref[...], preferred_element_type=jnp.float32)
    for q in range(NQ):
        o_ref[q] = dis[:, None] * hw[:, q * HQ:(q + 1) * HQ]


def _fin_kernel(a0, a1, a2, a3, y0, y1, y2, y3, d0, d1, b_ref, z_ref):
    dis = _dis(d0, d1)
    agg = jnp.concatenate([a0[0], a1[0], a2[0], a3[0]], axis=1)
    yy = jnp.concatenate([y0[0], y1[0], y2[0], y3[0]], axis=1)
    z_ref[...] = dis[:, None] * (agg + yy) + b_ref[...][None, :]


def _deg_specs():
    return [
        pl.BlockSpec((RB, 16), lambda i: (i, 0)),
        pl.BlockSpec((RB, 16), lambda i: (i + NBLK, 0)),
    ]


def _quarter_specs():
    return [
        pl.BlockSpec((1, RB, HQ), lambda i, k=k: (k, i, 0))
        for k in range(NQ)
    ]


_pre_call = pl.pallas_call(
    _pre_kernel,
    grid=(NBLK,),
    in_specs=_deg_specs() + [
        pl.BlockSpec((RB, D), lambda i: (i, 0)),
        pl.BlockSpec((D, D), lambda i: (0, 0)),
    ],
    out_specs=pl.BlockSpec((NQ, RB, HQ), lambda i: (0, i, 0)),
    out_shape=jax.ShapeDtypeStruct((NQ, N_PAD, HQ), jnp.float32),
)

_mid_call = pl.pallas_call(
    _mid_kernel,
    grid=(NBLK,),
    in_specs=_quarter_specs() + _quarter_specs() + _deg_specs() + [
        pl.BlockSpec((D,), lambda i: (0,)),
        pl.BlockSpec((D, D), lambda i: (0, 0)),
    ],
    out_specs=pl.BlockSpec((NQ, RB, HQ), lambda i: (0, i, 0)),
    out_shape=jax.ShapeDtypeStruct((NQ, N_PAD, HQ), jnp.float32),
)

_fin_call = pl.pallas_call(
    _fin_kernel,
    grid=(NBLK,),
    in_specs=_quarter_specs() + _quarter_specs() + _deg_specs() + [
        pl.BlockSpec((D,), lambda i: (0,)),
    ],
    out_specs=pl.BlockSpec((RB, D), lambda i: (i, 0)),
    out_shape=jax.ShapeDtypeStruct((N_PAD, D), jnp.float32),
)


def kernel(node_feature, edge_index, W1, b1, W2, b2):
    src = edge_index[0]
    dst = edge_index[1]
    pad = E_PAD - E
    trash = N + (jnp.arange(pad, dtype=jnp.int32) % (N_PAD - N))
    src_pad = jnp.concatenate([src, trash])
    dst_pad = jnp.concatenate([dst, trash])
    dstd = dst_pad.reshape(NC * NS * KD, CD)
    srcx = jnp.concatenate(
        [src_pad + q * N_PAD for q in range(NQ)]).reshape(NQ * NS * KA, CA)
    dstx = dst_pad.reshape(NS * KA, CA)
    xp = jnp.pad(node_feature, ((0, N_PAD - N), (0, 0)))
    ones16 = jnp.ones((CD, 16), jnp.float32)
    zeros16 = jnp.zeros((RPT, 16), jnp.float32)
    zerosQ = jnp.zeros((RPT, HQ), jnp.float32)

    deg2 = _deg_call(dstd, ones16, zeros16)

    y1 = _pre_call(deg2, deg2, xp, W1)
    agg1 = _agg_call(y1.reshape(NQ * N_PAD, HQ), srcx, dstx, zerosQ)
    a1 = agg1.reshape(NQ, N_PAD, HQ)
    y2 = _mid_call(a1, a1, a1, a1, y1, y1, y1, y1, deg2, deg2, b1, W2)
    agg2 = _agg_call(y2.reshape(NQ * N_PAD, HQ), srcx, dstx, zerosQ)
    a2 = agg2.reshape(NQ, N_PAD, HQ)
    z2 = _fin_call(a2, a2, a2, a2, y2, y2, y2, y2, deg2, deg2, b2)
    return z2[:N]


# speedup vs baseline: 1.0395x; 1.0003x over previous
import functools

import jax
import jax.numpy as jnp
from jax import lax
from jax.experimental import pallas as pl
from jax.experimental.pallas import tpu as pltpu
from jax.experimental.pallas import tpu_sc as plsc

N = 10000
E = 160000
D = 256

NC = 2
NS = 16
NQ = 4
HQ = D // NQ
N_PAD = 10240
E_PAD = 163840
CA = 128
KA = E_PAD // (NS * CA)
CD = 128
KD = E_PAD // (NC * NS * CD)
RPT = N_PAD // NS
NBUF = 8
RB = 1024
NBLK = N_PAD // RB

_mesh = plsc.VectorSubcoreMesh(core_axis_name="c", subcore_axis_name="s",
                               num_cores=NC, num_subcores=NS)
_sc_params = pltpu.CompilerParams(use_tc_tiling_on_sc=False)


def _deg_body(dstd, ones_hbm, zeros_hbm, out, idx_v, ones_v, sem, acc):
    c = lax.axis_index("c")
    s = lax.axis_index("s")
    pltpu.sync_copy(ones_hbm, ones_v)
    pltpu.sync_copy(zeros_hbm, acc.at[pl.ds(s * RPT, RPT)])
    plsc.subcore_barrier()
    pltpu.sync_copy(dstd.at[pl.ds((c * NS + s) * KD, KD)], idx_v)

    def step(i, carry):
        for b in range(4):
            j = i * 4 + b
            pltpu.async_copy(ones_v, acc.at[idx_v.at[j]], sem, add=True)
        for b in range(4):
            j = i * 4 + b
            pltpu.make_async_copy(ones_v, acc.at[idx_v.at[j]], sem).wait()
        return carry

    lax.fori_loop(0, KD // 4, step, 0)
    plsc.subcore_barrier()
    pltpu.sync_copy(acc.at[pl.ds(s * RPT, RPT)],
                    out.at[pl.ds(c * N_PAD + s * RPT, RPT)])


_deg_call = functools.partial(
    pl.kernel,
    out_type=jax.ShapeDtypeStruct((NC * N_PAD, 16), jnp.float32),
    mesh=_mesh,
    compiler_params=_sc_params,
    scratch_types=[
        pltpu.VMEM((KD, CD), jnp.int32),
        pltpu.VMEM((CD, 16), jnp.float32),
        pltpu.SemaphoreType.DMA,
        pltpu.VMEM_SHARED((N_PAD, 16), jnp.float32),
    ],
)(_deg_body)


def _agg_body(yflat, srcx, dstx, zeros_hbm, out, sidx_v, didx_v, rows_v, acc,
              *sems):
    gsems = sems[:NBUF]
    ssems = sems[NBUF:]
    c = lax.axis_index("c")
    s = lax.axis_index("s")

    pltpu.sync_copy(dstx.at[pl.ds(s * KA, KA)], didx_v)

    def fire_gather(j, b):
        pltpu.async_copy(yflat.at[sidx_v.at[j]], rows_v.at[b], gsems[b])

    def wait_gather(j, b):
        pltpu.make_async_copy(yflat.at[sidx_v.at[j]], rows_v.at[b],
                              gsems[b]).wait()

    def fire_scatter(j, b):
        pltpu.async_copy(rows_v.at[b], acc.at[didx_v.at[j]], ssems[b],
                         add=True)

    def wait_scatter(j, b):
        pltpu.make_async_copy(rows_v.at[b], acc.at[didx_v.at[j]],
                              ssems[b]).wait()

    for p in range(NQ // NC):
        q = c * (NQ // NC) + p
        pltpu.sync_copy(zeros_hbm, rows_v.at[0])
        for z in range(RPT // CA):
            pltpu.sync_copy(rows_v.at[0], acc.at[pl.ds(s * RPT + z * CA, CA)])
        pltpu.sync_copy(srcx.at[pl.ds((q * NS + s) * KA, KA)], sidx_v)
        plsc.subcore_barrier()

        for b in range(NBUF):
            fire_gather(b, b)
        steps = KA // NBUF

        def step(i, carry):
            for b in range(NBUF):
                j = i * NBUF + b
                wait_gather(j, b)
                fire_scatter(j, b)
            for b in range(NBUF):
                j = i * NBUF + b
                wait_scatter(j, b)
                fire_gather(j + NBUF, b)
            return carry

        lax.fori_loop(0, steps - 1, step, 0)
        for b in range(NBUF):
            j = (steps - 1) * NBUF + b
            wait_gather(j, b)
            fire_scatter(j, b)
        for b in range(NBUF):
            j = (steps - 1) * NBUF + b
            wait_scatter(j, b)
        plsc.subcore_barrier()
        pltpu.sync_copy(acc.at[pl.ds(s * RPT, RPT)],
                        out.at[pl.ds(q * N_PAD + s * RPT, RPT)])


_agg_call = functools.partial(
    pl.kernel,
    out_type=jax.ShapeDtypeStruct((NQ * N_PAD, HQ), jnp.float32),
    mesh=_mesh,
    compiler_params=_sc_params,
    scratch_types=[
        pltpu.VMEM((KA, CA), jnp.int32),
        pltpu.VMEM((KA, CA), jnp.int32),
        pltpu.VMEM((NBUF, CA, HQ), jnp.float32),
        pltpu.VMEM_SHARED((N_PAD, HQ), jnp.float32),
    ] + [pltpu.SemaphoreType.DMA] * (2 * NBUF),
)(_agg_body)


def _dis(d0_ref, d1_ref):
    deg = d0_ref[:, 0] + d1_ref[:, 0] + 1.0
    return lax.rsqrt(deg)


def _pre_kernel(d0, d1, x_ref, w_ref, o_ref):
    dis = _dis(d0, d1)
    xw = jnp.dot(x_ref[...], w_ref[...], preferred_element_type=jnp.float32)
    for q in range(NQ):
        o_ref[q] = dis[:, None] * xw[:, q * HQ:(q + 1) * HQ]


def _mid_kernel(a0, a1, a2, a3, y0, y1, y2, y3, d0, d1, b_ref, w_ref, o_ref):
    dis = _dis(d0, d1)
    agg = jnp.concatenate([a0[0], a1[0], a2[0], a3[0]], axis=1)
    yy = jnp.concatenate([y0[0], y1[0], y2[0], y3[0]], axis=1)
    hid = jnp.maximum(dis[:, None] * (agg + yy) + b_ref[...][None, :], 0.0)
    hw = jnp.dot(hid, w_ref[...], preferred_element_type=jnp.float32)
    for q in range(NQ):
        o_ref[q] = dis[:, None] * hw[:, q * HQ:(q + 1) * HQ]


def _fin_kernel(a0, a1, a2, a3, y0, y1, y2, y3, d0, d1, b_ref, z_ref):
    dis = _dis(d0, d1)
    agg = jnp.concatenate([a0[0], a1[0], a2[0], a3[0]], axis=1)
    yy = jnp.concatenate([y0[0], y1[0], y2[0], y3[0]], axis=1)
    z_ref[...] = dis[:, None] * (agg + yy) + b_ref[...][None, :]


def _deg_specs():
    return [
        pl.BlockSpec((RB, 16), lambda i: (i, 0)),
        pl.BlockSpec((RB, 16), lambda i: (i + NBLK, 0)),
    ]


def _quarter_specs():
    return [
        pl.BlockSpec((1, RB, HQ), lambda i, k=k: (k, i, 0))
        for k in range(NQ)
    ]


_pre_call = pl.pallas_call(
    _pre_kernel,
    grid=(NBLK,),
    in_specs=_deg_specs() + [
        pl.BlockSpec((RB, D), lambda i: (i, 0)),
        pl.BlockSpec((D, D), lambda i: (0, 0)),
    ],
    out_specs=pl.BlockSpec((NQ, RB, HQ), lambda i: (0, i, 0)),
    out_shape=jax.ShapeDtypeStruct((NQ, N_PAD, HQ), jnp.float32),
)

_mid_call = pl.pallas_call(
    _mid_kernel,
    grid=(NBLK,),
    in_specs=_quarter_specs() + _quarter_specs() + _deg_specs() + [
        pl.BlockSpec((D,), lambda i: (0,)),
        pl.BlockSpec((D, D), lambda i: (0, 0)),
    ],
    out_specs=pl.BlockSpec((NQ, RB, HQ), lambda i: (0, i, 0)),
    out_shape=jax.ShapeDtypeStruct((NQ, N_PAD, HQ), jnp.float32),
)

_fin_call = pl.pallas_call(
    _fin_kernel,
    grid=(NBLK,),
    in_specs=_quarter_specs() + _quarter_specs() + _deg_specs() + [
        pl.BlockSpec((D,), lambda i: (0,)),
    ],
    out_specs=pl.BlockSpec((RB, D), lambda i: (i, 0)),
    out_shape=jax.ShapeDtypeStruct((N_PAD, D), jnp.float32),
)


def kernel(node_feature, edge_index, W1, b1, W2, b2):
    src = edge_index[0]
    dst = edge_index[1]
    pad = E_PAD - E
    trash = N + (jnp.arange(pad, dtype=jnp.int32) % (N_PAD - N))
    src_pad = jnp.concatenate([src, trash])
    dst_pad = jnp.concatenate([dst, trash])
    dstd = dst_pad.reshape(NC * NS * KD, CD)
    srcx = jnp.concatenate(
        [src_pad + q * N_PAD for q in range(NQ)]).reshape(NQ * NS * KA, CA)
    dstx = dst_pad.reshape(NS * KA, CA)
    xp = jnp.pad(node_feature, ((0, N_PAD - N), (0, 0)))
    ones16 = jnp.ones((CD, 16), jnp.float32)
    zeros16 = jnp.zeros((RPT, 16), jnp.float32)
    zerosQ = jnp.zeros((CA, HQ), jnp.float32)

    deg2 = _deg_call(dstd, ones16, zeros16)

    y1 = _pre_call(deg2, deg2, xp, W1)
    agg1 = _agg_call(y1.reshape(NQ * N_PAD, HQ), srcx, dstx, zerosQ)
    a1 = agg1.reshape(NQ, N_PAD, HQ)
    y2 = _mid_call(a1, a1, a1, a1, y1, y1, y1, y1, deg2, deg2, b1, W2)
    agg2 = _agg_call(y2.reshape(NQ * N_PAD, HQ), srcx, dstx, zerosQ)
    a2 = agg2.reshape(NQ, N_PAD, HQ)
    z2 = _fin_call(a2, a2, a2, a2, y2, y2, y2, y2, deg2, deg2, b2)
    return z2[:N]


# speedup vs baseline: 1.0412x; 1.0016x over previous
import functools

import jax
import jax.numpy as jnp
from jax import lax
from jax.experimental import pallas as pl
from jax.experimental.pallas import tpu as pltpu
from jax.experimental.pallas import tpu_sc as plsc

N = 10000
E = 160000
D = 256

NC = 2
NS = 16
NQ = 4
HQ = D // NQ
N_PAD = 10240
E_PAD = 163840
CA = 128
KA = E_PAD // (NS * CA)
CD = 128
KD = E_PAD // (NC * NS * CD)
RPT = N_PAD // NS
NBUF = 8
RB = 1024
NBLK = N_PAD // RB

_mesh = plsc.VectorSubcoreMesh(core_axis_name="c", subcore_axis_name="s",
                               num_cores=NC, num_subcores=NS)
_sc_params = pltpu.CompilerParams(use_tc_tiling_on_sc=False)


def _deg_body(dstd, ones_hbm, zeros_hbm, out, idx_v, ones_v, sem, acc):
    c = lax.axis_index("c")
    s = lax.axis_index("s")
    pltpu.sync_copy(ones_hbm, ones_v)
    pltpu.sync_copy(zeros_hbm, acc.at[pl.ds(s * RPT, RPT)])
    plsc.subcore_barrier()
    pltpu.sync_copy(dstd.at[pl.ds((c * NS + s) * KD, KD)], idx_v)

    def step(i, carry):
        for b in range(4):
            j = i * 4 + b
            pltpu.async_copy(ones_v, acc.at[idx_v.at[j]], sem, add=True)
        for b in range(4):
            j = i * 4 + b
            pltpu.make_async_copy(ones_v, acc.at[idx_v.at[j]], sem).wait()
        return carry

    lax.fori_loop(0, KD // 4, step, 0)
    plsc.subcore_barrier()
    pltpu.sync_copy(acc.at[pl.ds(s * RPT, RPT)],
                    out.at[pl.ds(c * N_PAD + s * RPT, RPT)])


_deg_call = functools.partial(
    pl.kernel,
    out_type=jax.ShapeDtypeStruct((NC * N_PAD, 16), jnp.float32),
    mesh=_mesh,
    compiler_params=_sc_params,
    scratch_types=[
        pltpu.VMEM((KD, CD), jnp.int32),
        pltpu.VMEM((CD, 16), jnp.float32),
        pltpu.SemaphoreType.DMA,
        pltpu.VMEM_SHARED((N_PAD, 16), jnp.float32),
    ],
)(_deg_body)


def _agg_body(yflat, srcx, dstx, zeros_hbm, out, sidx_v, didx_v, rows_v, acc,
              *sems):
    gsems = sems[:NBUF]
    ssems = sems[NBUF:]
    c = lax.axis_index("c")
    s = lax.axis_index("s")

    pltpu.sync_copy(dstx.at[pl.ds(s * KA, KA)], didx_v)

    def fire_gather(j, b):
        pltpu.async_copy(yflat.at[sidx_v.at[j]], rows_v.at[b], gsems[b])

    def wait_gather(j, b):
        pltpu.make_async_copy(yflat.at[sidx_v.at[j]], rows_v.at[b],
                              gsems[b]).wait()

    def fire_scatter(j, b):
        pltpu.async_copy(rows_v.at[b], acc.at[didx_v.at[j]], ssems[b],
                         add=True)

    def wait_scatter(j, b):
        pltpu.make_async_copy(rows_v.at[b], acc.at[didx_v.at[j]],
                              ssems[b]).wait()

    for p in range(NQ // NC):
        q = c * (NQ // NC) + p
        pltpu.sync_copy(zeros_hbm, rows_v.at[0])
        for z in range(RPT // CA):
            pltpu.sync_copy(rows_v.at[0], acc.at[pl.ds(s * RPT + z * CA, CA)])
        pltpu.sync_copy(srcx.at[pl.ds((q * NS + s) * KA, KA)], sidx_v)
        plsc.subcore_barrier()

        for b in range(NBUF):
            fire_gather(b, b)
        steps = KA // NBUF

        def step(i, carry):
            for b in range(NBUF):
                j = i * NBUF + b
                wait_gather(j, b)
                fire_scatter(j, b)
            for b in range(NBUF):
                j = i * NBUF + b
                wait_scatter(j, b)
                fire_gather(j + NBUF, b)
            return carry

        lax.fori_loop(0, steps - 1, step, 0)
        for b in range(NBUF):
            j = (steps - 1) * NBUF + b
            wait_gather(j, b)
            fire_scatter(j, b)
        for b in range(NBUF):
            j = (steps - 1) * NBUF + b
            wait_scatter(j, b)
        plsc.subcore_barrier()
        pltpu.sync_copy(acc.at[pl.ds(s * RPT, RPT)],
                        out.at[pl.ds(q * N_PAD + s * RPT, RPT)])


_agg_call = functools.partial(
    pl.kernel,
    out_type=jax.ShapeDtypeStruct((NQ * N_PAD, HQ), jnp.float32),
    mesh=_mesh,
    compiler_params=_sc_params,
    scratch_types=[
        pltpu.VMEM((KA, CA), jnp.int32),
        pltpu.VMEM((KA, CA), jnp.int32),
        pltpu.VMEM((NBUF, CA, HQ), jnp.float32),
        pltpu.VMEM_SHARED((N_PAD, HQ), jnp.float32),
    ] + [pltpu.SemaphoreType.DMA] * (2 * NBUF),
)(_agg_body)


def _dis(d0_ref, d1_ref):
    deg = d0_ref[:, 0] + d1_ref[:, 0] + 1.0
    return lax.rsqrt(deg)


def _mm_kernel(x_ref, w_ref, o_ref):
    o_ref[...] = jnp.dot(x_ref[...], w_ref[...],
                         preferred_element_type=jnp.float32)


def _scale_kernel(d0, d1, xw_ref, o_ref):
    dis = _dis(d0, d1)
    xw = xw_ref[...]
    for q in range(NQ):
        o_ref[q] = dis[:, None] * xw[:, q * HQ:(q + 1) * HQ]


def _mid_kernel(a0, a1, a2, a3, y0, y1, y2, y3, d0, d1, b_ref, w_ref, o_ref):
    dis = _dis(d0, d1)
    agg = jnp.concatenate([a0[0], a1[0], a2[0], a3[0]], axis=1)
    yy = jnp.concatenate([y0[0], y1[0], y2[0], y3[0]], axis=1)
    hid = jnp.maximum(dis[:, None] * (agg + yy) + b_ref[...][None, :], 0.0)
    hw = jnp.dot(hid, w_ref[...], preferred_element_type=jnp.float32)
    for q in range(NQ):
        o_ref[q] = dis[:, None] * hw[:, q * HQ:(q + 1) * HQ]


def _fin_kernel(a0, a1, a2, a3, y0, y1, y2, y3, d0, d1, b_ref, z_ref):
    dis = lax.rsqrt(d0[0][:, 0] + d1[0][:, 0] + 1.0)
    agg = jnp.concatenate([a0[0], a1[0], a2[0], a3[0]], axis=1)
    yy = jnp.concatenate([y0[0], y1[0], y2[0], y3[0]], axis=1)
    z_ref[...] = dis[:, None] * (agg + yy) + b_ref[...][None, :]


def _deg_specs():
    return [
        pl.BlockSpec((RB, 16), lambda i: (i, 0)),
        pl.BlockSpec((RB, 16), lambda i: (i + NBLK, 0)),
    ]


def _quarter_specs():
    return [
        pl.BlockSpec((1, RB, HQ), lambda i, k=k: (k, i, 0))
        for k in range(NQ)
    ]


_mm_call = pl.pallas_call(
    _mm_kernel,
    grid=(NBLK,),
    in_specs=[
        pl.BlockSpec((RB, D), lambda i: (i, 0)),
        pl.BlockSpec((D, D), lambda i: (0, 0)),
    ],
    out_specs=pl.BlockSpec((RB, D), lambda i: (i, 0)),
    out_shape=jax.ShapeDtypeStruct((N_PAD, D), jnp.float32),
)

_scale_call = pl.pallas_call(
    _scale_kernel,
    grid=(NBLK,),
    in_specs=_deg_specs() + [
        pl.BlockSpec((RB, D), lambda i: (i, 0)),
    ],
    out_specs=pl.BlockSpec((NQ, RB, HQ), lambda i: (0, i, 0)),
    out_shape=jax.ShapeDtypeStruct((NQ, N_PAD, HQ), jnp.float32),
)

_mid_call = pl.pallas_call(
    _mid_kernel,
    grid=(NBLK,),
    in_specs=_quarter_specs() + _quarter_specs() + _deg_specs() + [
        pl.BlockSpec((D,), lambda i: (0,)),
        pl.BlockSpec((D, D), lambda i: (0, 0)),
    ],
    out_specs=pl.BlockSpec((NQ, RB, HQ), lambda i: (0, i, 0)),
    out_shape=jax.ShapeDtypeStruct((NQ, N_PAD, HQ), jnp.float32),
)

RBF = 1000

_fin_call = pl.pallas_call(
    _fin_kernel,
    grid=(N // RBF,),
    in_specs=[
        pl.BlockSpec((1, RBF, HQ), lambda i, k=k: (k, i, 0))
        for k in range(NQ)
    ] * 2 + [
        pl.BlockSpec((1, RBF, 16), lambda i: (0, i, 0)),
        pl.BlockSpec((1, RBF, 16), lambda i: (1, i, 0)),
    ] + [
        pl.BlockSpec((D,), lambda i: (0,)),
    ],
    out_specs=pl.BlockSpec((RBF, D), lambda i: (i, 0)),
    out_shape=jax.ShapeDtypeStruct((N, D), jnp.float32),
)


def kernel(node_feature, edge_index, W1, b1, W2, b2):
    src = edge_index[0]
    dst = edge_index[1]
    pad = E_PAD - E
    trash = N + (jnp.arange(pad, dtype=jnp.int32) % (N_PAD - N))
    src_pad = jnp.concatenate([src, trash])
    dst_pad = jnp.concatenate([dst, trash])
    dstd = dst_pad.reshape(NC * NS * KD, CD)
    srcx = jnp.concatenate(
        [src_pad + q * N_PAD for q in range(NQ)]).reshape(NQ * NS * KA, CA)
    dstx = dst_pad.reshape(NS * KA, CA)
    xp = jnp.pad(node_feature, ((0, N_PAD - N), (0, 0)))
    ones16 = jnp.ones((CD, 16), jnp.float32)
    zeros16 = jnp.zeros((RPT, 16), jnp.float32)
    zerosQ = jnp.zeros((CA, HQ), jnp.float32)

    deg2 = _deg_call(dstd, ones16, zeros16)
    xw1 = _mm_call(xp, W1)

    y1 = _scale_call(deg2, deg2, xw1)
    agg1 = _agg_call(y1.reshape(NQ * N_PAD, HQ), srcx, dstx, zerosQ)
    a1 = agg1.reshape(NQ, N_PAD, HQ)
    y2 = _mid_call(a1, a1, a1, a1, y1, y1, y1, y1, deg2, deg2, b1, W2)
    agg2 = _agg_call(y2.reshape(NQ * N_PAD, HQ), srcx, dstx, zerosQ)
    a2 = agg2.reshape(NQ, N_PAD, HQ)
    deg3 = deg2.reshape(2, N_PAD, 16)
    return _fin_call(a2, a2, a2, a2, y2, y2, y2, y2, deg3, deg3, b2)
